# baseline (device time: 10332 ns/iter reference)
import jax
import jax.numpy as jnp
from jax import lax
from jax.experimental import pallas as pl
from jax.experimental.pallas import tpu as pltpu

BLOCKS = 4
LANES = 128


def kernel(x):
    m, n = x.shape
    mb = m // BLOCKS
    pk = m // LANES
    pb = pk // BLOCKS
    half = pk // 2

    def body(x_hbm, out_ref, xb, send_buf, recv_buf, in_sems,
             send_sems, recv_sems):
        my_x = lax.axis_index("x")
        my_y = lax.axis_index("y")
        nbr = (my_x, 1 - my_y)

        barrier_sem = pltpu.get_barrier_semaphore()
        pl.semaphore_signal(
            barrier_sem, inc=1,
            device_id=nbr, device_id_type=pl.DeviceIdType.MESH,
        )

        def in_copy(b):
            return pltpu.make_async_copy(
                x_hbm.at[pl.ds(b * mb, mb), :],
                xb.at[b],
                in_sems.at[b],
            )

        def half_rdma(h):
            return pltpu.make_async_remote_copy(
                src_ref=send_buf.at[pl.ds(h * half, half), :],
                dst_ref=recv_buf.at[pl.ds(h * half, half), :],
                send_sem=send_sems.at[h],
                recv_sem=recv_sems.at[h],
                device_id=nbr,
                device_id_type=pl.DeviceIdType.MESH,
            )

        for b in range(BLOCKS):
            in_copy(b).start()
        for b in range(BLOCKS):
            in_copy(b).wait()
            blk = jnp.max(xb[b], axis=1)
            send_buf[pl.ds(b * pb, pb), :] = blk.reshape(pb, LANES)
            if b == BLOCKS // 2 - 1:
                pl.semaphore_wait(barrier_sem, 1)
                half_rdma(0).start()

        r1 = half_rdma(1)
        r1.start()
        r0 = half_rdma(0)
        r0.wait_send()
        r0.wait_recv()
        r1.wait_send()
        r1.wait_recv()
        out_ref[...] = jnp.maximum(send_buf[...], recv_buf[...])

    x = pltpu.with_memory_space_constraint(x, pltpu.MemorySpace.HBM)
    packed = pl.pallas_call(
        body,
        out_shape=jax.ShapeDtypeStruct((pk, LANES), x.dtype),
        in_specs=[pl.BlockSpec(memory_space=pltpu.MemorySpace.HBM)],
        out_specs=pl.BlockSpec(memory_space=pltpu.VMEM),
        scratch_shapes=[
            pltpu.VMEM((BLOCKS, mb, n), x.dtype),
            pltpu.VMEM((pk, LANES), x.dtype),
            pltpu.VMEM((pk, LANES), x.dtype),
            pltpu.SemaphoreType.DMA((BLOCKS,)),
            pltpu.SemaphoreType.DMA((2,)),
            pltpu.SemaphoreType.DMA((2,)),
        ],
        compiler_params=pltpu.CompilerParams(collective_id=0),
    )(x)
    return packed.reshape(m, 1)


# device time: 10255 ns/iter; 1.0075x vs baseline; 1.0075x over previous
import jax
import jax.numpy as jnp
from jax import lax
from jax.experimental import pallas as pl
from jax.experimental.pallas import tpu as pltpu

BLOCKS = 8
LANES = 128


def kernel(x):
    m, n = x.shape
    mb = m // BLOCKS
    pk = m // LANES
    pb = pk // BLOCKS
    half = pk // 2

    def body(x_hbm, out_ref, xb, send_buf, recv_buf, in_sems,
             send_sems, recv_sems):
        my_x = lax.axis_index("x")
        my_y = lax.axis_index("y")
        nbr = (my_x, 1 - my_y)

        barrier_sem = pltpu.get_barrier_semaphore()
        pl.semaphore_signal(
            barrier_sem, inc=1,
            device_id=nbr, device_id_type=pl.DeviceIdType.MESH,
        )

        def in_copy(b):
            return pltpu.make_async_copy(
                x_hbm.at[pl.ds(b * mb, mb), :],
                xb.at[b],
                in_sems.at[b],
            )

        def half_rdma(h):
            return pltpu.make_async_remote_copy(
                src_ref=send_buf.at[pl.ds(h * half, half), :],
                dst_ref=recv_buf.at[pl.ds(h * half, half), :],
                send_sem=send_sems.at[h],
                recv_sem=recv_sems.at[h],
                device_id=nbr,
                device_id_type=pl.DeviceIdType.MESH,
            )

        for b in range(BLOCKS):
            in_copy(b).start()
        for b in range(BLOCKS):
            in_copy(b).wait()
            blk = jnp.max(xb[b], axis=1)
            send_buf[pl.ds(b * pb, pb), :] = blk.reshape(pb, LANES)
            if b == BLOCKS // 2 - 1:
                pl.semaphore_wait(barrier_sem, 1)
                half_rdma(0).start()

        r1 = half_rdma(1)
        r1.start()
        r0 = half_rdma(0)
        r0.wait_send()
        r0.wait_recv()
        r1.wait_send()
        r1.wait_recv()
        out_ref[...] = jnp.maximum(send_buf[...], recv_buf[...])

    x = pltpu.with_memory_space_constraint(x, pltpu.MemorySpace.HBM)
    packed = pl.pallas_call(
        body,
        out_shape=jax.ShapeDtypeStruct((pk, LANES), x.dtype),
        in_specs=[pl.BlockSpec(memory_space=pltpu.MemorySpace.HBM)],
        out_specs=pl.BlockSpec(memory_space=pltpu.VMEM),
        scratch_shapes=[
            pltpu.VMEM((BLOCKS, mb, n), x.dtype),
            pltpu.VMEM((pk, LANES), x.dtype),
            pltpu.VMEM((pk, LANES), x.dtype),
            pltpu.SemaphoreType.DMA((BLOCKS,)),
            pltpu.SemaphoreType.DMA((2,)),
            pltpu.SemaphoreType.DMA((2,)),
        ],
        compiler_params=pltpu.CompilerParams(collective_id=0),
    )(x)
    return packed.reshape(m, 1)
